# baseline (speedup 1.0000x reference)
import functools

import jax
import jax.numpy as jnp
from jax import lax
from jax.experimental import pallas as pl
from jax.experimental.pallas import tpu as pltpu
from jax.experimental.pallas import tpu_sc as plsc

B, S, D = 1, 2048, 768
E, K, F = 8, 2, 1024
T = B * S
A = T * K
BM = 256
BM_SHIFT = 8
PA = A + E * BM
G = PA // BM
BR = 1024
FT = 512

NC, NS, L = 2, 16, 16
NW = NC * NS
APW = A // NW
TPW = T // NW
CPW = APW // 2


def _router_body(x_ref, wg_ref, eidx_ref, wtop_ref, rank_ref, poff_ref,
                 cnt_scr):
    t = pl.program_id(0)

    @pl.when(t == 0)
    def _():
        cnt_scr[...] = jnp.zeros_like(cnt_scr)

    x = x_ref[...]
    wg = wg_ref[...]
    logits = lax.dot_general(x, wg, (((1,), (1,)), ((), ())),
                             preferred_element_type=jnp.float32)
    iota = lax.broadcasted_iota(jnp.int32, (BR, E), 1)
    big = jnp.int32(2**30)
    m1 = jnp.max(logits, axis=1, keepdims=True)
    i1 = jnp.min(jnp.where(logits == m1, iota, big), axis=1, keepdims=True)
    masked = jnp.where(iota == i1, -jnp.inf, logits)
    m2 = jnp.max(masked, axis=1, keepdims=True)
    i2 = jnp.min(jnp.where(masked == m2, iota, big), axis=1, keepdims=True)
    e2 = jnp.exp(m2 - m1)
    w1 = 1.0 / (1.0 + e2)
    w2 = e2 / (1.0 + e2)

    oh0 = (iota == i1).astype(jnp.float32)
    oh1 = (iota == i2).astype(jnp.float32)
    ohp = oh0 + oh1
    ri = lax.broadcasted_iota(jnp.int32, (BR, BR), 0)
    ci = lax.broadcasted_iota(jnp.int32, (BR, BR), 1)
    tri = (ci < ri).astype(jnp.float32)
    cum = lax.dot_general(tri, ohp, (((1,), (0,)), ((), ())),
                          preferred_element_type=jnp.float32)
    base = cnt_scr[...] + cum
    rank0 = jnp.sum(oh0 * base, axis=1, keepdims=True)
    rank1 = jnp.sum(oh1 * base, axis=1, keepdims=True)
    cnt_new = cnt_scr[...] + jnp.sum(ohp, axis=0, keepdims=True)
    cnt_scr[...] = cnt_new

    eidx_ref[...] = jnp.concatenate([i1, i2], axis=1)
    wtop_ref[...] = jnp.concatenate([w1, w2], axis=1)
    rank_ref[...] = jnp.concatenate([rank0, rank1], axis=1).astype(jnp.int32)
    pcnt = jnp.floor((cnt_new + (BM - 1)) * (1.0 / BM)) * BM
    ei = lax.broadcasted_iota(jnp.int32, (E, E), 0)
    ej = lax.broadcasted_iota(jnp.int32, (E, E), 1)
    stri = (ei < ej).astype(jnp.float32)
    poff = lax.dot_general(pcnt, stri, (((1,), (0,)), ((), ())),
                           preferred_element_type=jnp.float32)
    total = jnp.sum(pcnt, axis=1, keepdims=True)
    pad = jnp.zeros((1, 16 - E - 1), jnp.int32)
    poff_ref[...] = jnp.concatenate(
        [poff.astype(jnp.int32), total.astype(jnp.int32), pad], axis=1)


def _router(xf, Wgate):
    return pl.pallas_call(
        _router_body,
        grid=(T // BR,),
        in_specs=[
            pl.BlockSpec((BR, D), lambda t: (t, 0)),
            pl.BlockSpec((E, D), lambda t: (0, 0)),
        ],
        out_specs=[
            pl.BlockSpec((BR, K), lambda t: (t, 0)),
            pl.BlockSpec((BR, K), lambda t: (t, 0)),
            pl.BlockSpec((BR, K), lambda t: (t, 0)),
            pl.BlockSpec((1, 16), lambda t: (0, 0)),
        ],
        out_shape=[
            jax.ShapeDtypeStruct((T, K), jnp.int32),
            jax.ShapeDtypeStruct((T, K), jnp.float32),
            jax.ShapeDtypeStruct((T, K), jnp.int32),
            jax.ShapeDtypeStruct((1, 16), jnp.int32),
        ],
        scratch_shapes=[pltpu.VMEM((1, E), jnp.float32)],
    )(xf, Wgate)


@functools.cache
def _get_sc_dispatch():
    @functools.partial(
        pl.kernel,
        out_type=jax.ShapeDtypeStruct((PA, D), jnp.float32),
        mesh=plsc.VectorSubcoreMesh(core_axis_name="c", subcore_axis_name="s",
                                    num_cores=NC, num_subcores=NS),
        compiler_params=pltpu.CompilerParams(needs_layout_passes=False),
        scratch_types=[
            pltpu.VMEM((16,), jnp.int32),
            pltpu.VMEM((APW,), jnp.int32),
            pltpu.VMEM((APW,), jnp.int32),
            pltpu.VMEM((TPW,), jnp.int32),
            pltpu.VMEM((TPW,), jnp.int32),
            pltpu.VMEM((TPW, D), jnp.float32),
            pltpu.SemaphoreType.DMA,
            pltpu.SemaphoreType.DMA,
            pltpu.SemaphoreType.DMA,
        ],
    )
    def _sc_dispatch(xf_hbm, eflat_hbm, rflat_hbm, poff_hbm, xs_hbm,
                     poff_vm, e_vm, r_vm, pos0_vm, pos1_vm, rows_vm,
                     semg, sems0, sems1):
        wid = lax.axis_index("s") * NC + lax.axis_index("c")
        abase = wid * APW
        tbase = wid * TPW
        g = pltpu.async_copy(xf_hbm.at[pl.ds(tbase, TPW)], rows_vm, semg)
        pltpu.sync_copy(poff_hbm, poff_vm)
        pltpu.sync_copy(eflat_hbm.at[pl.ds(abase, APW)], e_vm)
        pltpu.sync_copy(rflat_hbm.at[pl.ds(abase, APW)], r_vm)
        for i in range(TPW // L):
            l_v = i * L + lax.iota(jnp.int32, L)
            for k, dst in ((0, pos0_vm), (1, pos1_vm)):
                a_v = 2 * l_v + k
                e_v = plsc.load_gather(e_vm, [a_v])
                r_v = plsc.load_gather(r_vm, [a_v])
                dst[pl.ds(i * L, L)] = plsc.load_gather(poff_vm, [e_v]) + r_v
        g.wait()
        s0 = pltpu.async_copy(rows_vm, xs_hbm.at[pos0_vm], sems0)
        s1 = pltpu.async_copy(rows_vm, xs_hbm.at[pos1_vm], sems1)
        s0.wait()
        s1.wait()

    return _sc_dispatch


def _clamp_b(b, poff_ref):
    return jnp.minimum(b, poff_ref[E] // BM - 1)


def _block_expert(b, poff_ref):
    e = jnp.int32(0)
    for i in range(1, E):
        e = e + jnp.where(b * BM >= poff_ref[i], 1, 0).astype(jnp.int32)
    return e


def _ffn_body(poff_ref, xs_ref, wg_ref, wu_ref, wd_ref, out_ref):
    b = pl.program_id(0)
    ft = pl.program_id(1)

    @pl.when(b * BM < poff_ref[E])
    def _():
        x = xs_ref[...]
        xg = lax.dot_general(x, wg_ref[0], (((1,), (1,)), ((), ())),
                             preferred_element_type=jnp.float32)
        xu = lax.dot_general(x, wu_ref[0], (((1,), (1,)), ((), ())),
                             preferred_element_type=jnp.float32)
        h = (xg * jax.nn.sigmoid(xg)) * xu
        part = lax.dot_general(h, wd_ref[0], (((1,), (1,)), ((), ())),
                               preferred_element_type=jnp.float32)

        @pl.when(ft == 0)
        def _():
            out_ref[...] = part

        @pl.when(ft != 0)
        def _():
            out_ref[...] = out_ref[...] + part


def _ffn_grouped(poff_flat, xs, Wg, Wu, Wd):
    grid_spec = pltpu.PrefetchScalarGridSpec(
        num_scalar_prefetch=1,
        grid=(G, F // FT),
        in_specs=[
            pl.BlockSpec((BM, D), lambda b, ft, p: (_clamp_b(b, p), 0)),
            pl.BlockSpec(
                (1, FT, D),
                lambda b, ft, p: (_block_expert(_clamp_b(b, p), p), ft, 0)),
            pl.BlockSpec(
                (1, FT, D),
                lambda b, ft, p: (_block_expert(_clamp_b(b, p), p), ft, 0)),
            pl.BlockSpec(
                (1, D, FT),
                lambda b, ft, p: (_block_expert(_clamp_b(b, p), p), 0, ft)),
        ],
        out_specs=pl.BlockSpec((BM, D), lambda b, ft, p: (_clamp_b(b, p), 0)),
    )
    return pl.pallas_call(
        _ffn_body,
        grid_spec=grid_spec,
        out_shape=jax.ShapeDtypeStruct((PA, D), jnp.float32),
    )(poff_flat, xs, Wg, Wu, Wd)


@functools.cache
def _get_sc_combine():
    @functools.partial(
        pl.kernel,
        out_type=jax.ShapeDtypeStruct((T, D), jnp.float32),
        mesh=plsc.VectorSubcoreMesh(core_axis_name="c", subcore_axis_name="s",
                                    num_cores=NC, num_subcores=NS),
        compiler_params=pltpu.CompilerParams(needs_layout_passes=False),
        scratch_types=[
            pltpu.VMEM((16,), jnp.int32),
            pltpu.VMEM((APW,), jnp.int32),
            pltpu.VMEM((APW,), jnp.int32),
            pltpu.VMEM((APW,), jnp.float32),
            pltpu.VMEM((APW,), jnp.int32),
            pltpu.VMEM((APW, D), jnp.float32),
            pltpu.SemaphoreType.DMA,
        ],
    )
    def _sc_combine(outp_hbm, eflat_hbm, rflat_hbm, wflat_hbm, poff_hbm,
                    y_hbm, poff_vm, e_vm, r_vm, w_vm, pos_vm, rows_vm, sem1):
        wid = lax.axis_index("s") * NC + lax.axis_index("c")
        abase = wid * APW
        tbase = wid * TPW
        pltpu.sync_copy(poff_hbm, poff_vm)
        pltpu.sync_copy(eflat_hbm.at[pl.ds(abase, APW)], e_vm)
        pltpu.sync_copy(rflat_hbm.at[pl.ds(abase, APW)], r_vm)
        pltpu.sync_copy(wflat_hbm.at[pl.ds(abase, APW)], w_vm)
        for i in range(APW // L):
            k = i // (TPW // L)
            l_v = (i % (TPW // L)) * L + lax.iota(jnp.int32, L)
            a_v = 2 * l_v + k
            e_v = plsc.load_gather(e_vm, [a_v])
            r_v = plsc.load_gather(r_vm, [a_v])
            pos_vm[pl.ds(i * L, L)] = plsc.load_gather(poff_vm, [e_v]) + r_v
        pltpu.async_copy(outp_hbm.at[pos_vm], rows_vm, sem1).wait()

        def body(l, carry):
            i0 = jnp.zeros((L,), jnp.int32) + 2 * l
            w0 = plsc.load_gather(w_vm, [i0])
            w1 = plsc.load_gather(w_vm, [i0 + 1])
            for v in range(D // L):
                sl = pl.ds(v * L, L)
                rows_vm[l, sl] = (rows_vm[l, sl] * w0
                                  + rows_vm[TPW + l, sl] * w1)
            return carry

        lax.fori_loop(0, TPW, body, 0)
        pltpu.sync_copy(rows_vm.at[pl.ds(0, TPW)], y_hbm.at[pl.ds(tbase, TPW)])

    return _sc_combine


def kernel(x, Wgate, Wg, Wu, Wd):
    xf = x.reshape(T, D)
    eidx, wtop, rank, poff16 = _router(xf, Wgate)

    eflat = eidx.reshape(A)
    rflat = rank.reshape(A)
    wflat = wtop.reshape(A)
    poff_flat = poff16.reshape(16)

    xs = _get_sc_dispatch()(xf, eflat, rflat, poff_flat)
    outp = _ffn_grouped(poff_flat, xs, Wg, Wu, Wd)
    y = _get_sc_combine()(outp, eflat, rflat, wflat, poff_flat)
    return y.reshape(B, S, D)


# speedup vs baseline: 1.2942x; 1.2942x over previous
import functools

import jax
import jax.numpy as jnp
from jax import lax
from jax.experimental import pallas as pl
from jax.experimental.pallas import tpu as pltpu
from jax.experimental.pallas import tpu_sc as plsc

B, S, D = 1, 2048, 768
E, K, F = 8, 2, 1024
T = B * S
A = T * K
BM = 256
PA = A + E * BM
G = PA // BM
BR = 1024

NC, NS, L = 2, 16, 16
NW = NC * NS
APW = A // NW
TPW = T // NW


def _router_body(x_ref, wg_ref, eidx_ref, wtop_ref, rank_ref, poff_ref,
                 cnt_scr):
    t = pl.program_id(0)

    @pl.when(t == 0)
    def _():
        cnt_scr[...] = jnp.zeros_like(cnt_scr)

    x = x_ref[...]
    wg = wg_ref[...]
    logits = lax.dot_general(x, wg, (((1,), (1,)), ((), ())),
                             preferred_element_type=jnp.float32)
    ex = jnp.exp(logits - jnp.max(logits, axis=1, keepdims=True))
    p = ex / jnp.sum(ex, axis=1, keepdims=True)
    iota = lax.broadcasted_iota(jnp.int32, (BR, E), 1)
    big = jnp.int32(2**30)
    m1 = jnp.max(p, axis=1, keepdims=True)
    i1 = jnp.min(jnp.where(p == m1, iota, big), axis=1, keepdims=True)
    masked = jnp.where(iota == i1, -jnp.float32(1.0), p)
    m2 = jnp.max(masked, axis=1, keepdims=True)
    i2 = jnp.min(jnp.where(masked == m2, iota, big), axis=1, keepdims=True)
    denom = m1 + m2 + jnp.float32(1e-20)
    w1 = m1 / denom
    w2 = m2 / denom

    oh0 = (iota == i1).astype(jnp.float32)
    oh1 = (iota == i2).astype(jnp.float32)
    ohp = oh0 + oh1
    ri = lax.broadcasted_iota(jnp.int32, (BR, BR), 0)
    ci = lax.broadcasted_iota(jnp.int32, (BR, BR), 1)
    tri = (ci < ri).astype(jnp.float32)
    cum = lax.dot_general(tri, ohp, (((1,), (0,)), ((), ())),
                          preferred_element_type=jnp.float32)
    base = cnt_scr[...] + cum
    rank0 = jnp.sum(oh0 * base, axis=1, keepdims=True)
    rank1 = jnp.sum(oh1 * base, axis=1, keepdims=True)
    cnt_new = cnt_scr[...] + jnp.sum(ohp, axis=0, keepdims=True)
    cnt_scr[...] = cnt_new

    eidx_ref[...] = jnp.concatenate([i1, i2], axis=1)
    wtop_ref[...] = jnp.concatenate([w1, w2], axis=1)
    rank_ref[...] = jnp.concatenate([rank0, rank1], axis=1).astype(jnp.int32)
    pcnt = jnp.floor((cnt_new + (BM - 1)) * (1.0 / BM)) * BM
    ei = lax.broadcasted_iota(jnp.int32, (E, E), 0)
    ej = lax.broadcasted_iota(jnp.int32, (E, E), 1)
    stri = (ei < ej).astype(jnp.float32)
    poff = lax.dot_general(pcnt, stri, (((1,), (0,)), ((), ())),
                           preferred_element_type=jnp.float32)
    total = jnp.sum(pcnt, axis=1, keepdims=True)
    pad = jnp.zeros((1, 16 - E - 1), jnp.int32)
    poff_ref[...] = jnp.concatenate(
        [poff.astype(jnp.int32), total.astype(jnp.int32), pad], axis=1)


def _router(xf, Wgate):
    return pl.pallas_call(
        _router_body,
        grid=(T // BR,),
        in_specs=[
            pl.BlockSpec((BR, D), lambda t: (t, 0)),
            pl.BlockSpec((E, D), lambda t: (0, 0)),
        ],
        out_specs=[
            pl.BlockSpec((BR, K), lambda t: (t, 0)),
            pl.BlockSpec((BR, K), lambda t: (t, 0)),
            pl.BlockSpec((BR, K), lambda t: (t, 0)),
            pl.BlockSpec((1, 16), lambda t: (0, 0)),
        ],
        out_shape=[
            jax.ShapeDtypeStruct((T, K), jnp.int32),
            jax.ShapeDtypeStruct((T, K), jnp.float32),
            jax.ShapeDtypeStruct((T, K), jnp.int32),
            jax.ShapeDtypeStruct((1, 16), jnp.int32),
        ],
        scratch_shapes=[pltpu.VMEM((1, E), jnp.float32)],
    )(xf, Wgate)


@functools.cache
def _get_sc_dispatch():
    @functools.partial(
        pl.kernel,
        out_type=jax.ShapeDtypeStruct((PA, D), jnp.float32),
        mesh=plsc.VectorSubcoreMesh(core_axis_name="c", subcore_axis_name="s",
                                    num_cores=NC, num_subcores=NS),
        compiler_params=pltpu.CompilerParams(needs_layout_passes=False),
        scratch_types=[
            pltpu.VMEM((16,), jnp.int32),
            pltpu.VMEM((APW,), jnp.int32),
            pltpu.VMEM((APW,), jnp.int32),
            pltpu.VMEM((TPW,), jnp.int32),
            pltpu.VMEM((TPW,), jnp.int32),
            pltpu.VMEM((TPW, D), jnp.float32),
            pltpu.SemaphoreType.DMA,
            pltpu.SemaphoreType.DMA,
            pltpu.SemaphoreType.DMA,
        ],
    )
    def _sc_dispatch(xf_hbm, eflat_hbm, rflat_hbm, poff_hbm, xs_hbm,
                     poff_vm, e_vm, r_vm, pos0_vm, pos1_vm, rows_vm,
                     semg, sems0, sems1):
        wid = lax.axis_index("s") * NC + lax.axis_index("c")
        abase = wid * APW
        tbase = wid * TPW
        g = pltpu.async_copy(xf_hbm.at[pl.ds(tbase, TPW)], rows_vm, semg)
        pltpu.sync_copy(poff_hbm, poff_vm)
        pltpu.sync_copy(eflat_hbm.at[pl.ds(abase, APW)], e_vm)
        pltpu.sync_copy(rflat_hbm.at[pl.ds(abase, APW)], r_vm)
        for i in range(TPW // L):
            l_v = i * L + lax.iota(jnp.int32, L)
            for k, dst in ((0, pos0_vm), (1, pos1_vm)):
                a_v = 2 * l_v + k
                e_v = plsc.load_gather(e_vm, [a_v])
                r_v = plsc.load_gather(r_vm, [a_v])
                dst[pl.ds(i * L, L)] = plsc.load_gather(poff_vm, [e_v]) + r_v
        g.wait()
        s0 = pltpu.async_copy(rows_vm, xs_hbm.at[pos0_vm], sems0)
        s1 = pltpu.async_copy(rows_vm, xs_hbm.at[pos1_vm], sems1)
        s0.wait()
        s1.wait()

    return _sc_dispatch


def _clamp_b(b, poff_ref):
    return jnp.minimum(b, poff_ref[E] // BM - 1)


def _block_expert(b, poff_ref):
    e = jnp.int32(0)
    for i in range(1, E):
        e = e + jnp.where(b * BM >= poff_ref[i], 1, 0).astype(jnp.int32)
    return e


def _ffn_body(poff_ref, xs_ref, wg_ref, wu_ref, wd_ref, out_ref):
    b = pl.program_id(0)

    @pl.when(b * BM < poff_ref[E])
    def _():
        x = xs_ref[...]
        xg = lax.dot_general(x, wg_ref[0], (((1,), (1,)), ((), ())),
                             preferred_element_type=jnp.float32)
        xu = lax.dot_general(x, wu_ref[0], (((1,), (1,)), ((), ())),
                             preferred_element_type=jnp.float32)
        h = (xg * jax.nn.sigmoid(xg)) * xu
        out_ref[...] = lax.dot_general(h, wd_ref[0],
                                       (((1,), (1,)), ((), ())),
                                       preferred_element_type=jnp.float32)


def _ffn_grouped(poff_flat, xs, Wg, Wu, Wd):
    grid_spec = pltpu.PrefetchScalarGridSpec(
        num_scalar_prefetch=1,
        grid=(G,),
        in_specs=[
            pl.BlockSpec((BM, D), lambda b, p: (_clamp_b(b, p), 0)),
            pl.BlockSpec((1, F, D),
                         lambda b, p: (_block_expert(_clamp_b(b, p), p), 0, 0)),
            pl.BlockSpec((1, F, D),
                         lambda b, p: (_block_expert(_clamp_b(b, p), p), 0, 0)),
            pl.BlockSpec((1, D, F),
                         lambda b, p: (_block_expert(_clamp_b(b, p), p), 0, 0)),
        ],
        out_specs=pl.BlockSpec((BM, D), lambda b, p: (_clamp_b(b, p), 0)),
    )
    return pl.pallas_call(
        _ffn_body,
        grid_spec=grid_spec,
        out_shape=jax.ShapeDtypeStruct((PA, D), jnp.float32),
    )(poff_flat, xs, Wg, Wu, Wd)


@functools.cache
def _get_sc_combine():
    @functools.partial(
        pl.kernel,
        out_type=jax.ShapeDtypeStruct((T, D), jnp.float32),
        mesh=plsc.VectorSubcoreMesh(core_axis_name="c", subcore_axis_name="s",
                                    num_cores=NC, num_subcores=NS),
        compiler_params=pltpu.CompilerParams(needs_layout_passes=False),
        scratch_types=[
            pltpu.VMEM((16,), jnp.int32),
            pltpu.VMEM((APW,), jnp.int32),
            pltpu.VMEM((APW,), jnp.int32),
            pltpu.VMEM((APW,), jnp.float32),
            pltpu.VMEM((APW,), jnp.int32),
            pltpu.VMEM((APW, D), jnp.float32),
            pltpu.SemaphoreType.DMA,
        ],
    )
    def _sc_combine(outp_hbm, eflat_hbm, rflat_hbm, wflat_hbm, poff_hbm,
                    y_hbm, poff_vm, e_vm, r_vm, w_vm, pos_vm, rows_vm, sem1):
        wid = lax.axis_index("s") * NC + lax.axis_index("c")
        abase = wid * APW
        tbase = wid * TPW
        pltpu.sync_copy(poff_hbm, poff_vm)
        pltpu.sync_copy(eflat_hbm.at[pl.ds(abase, APW)], e_vm)
        pltpu.sync_copy(rflat_hbm.at[pl.ds(abase, APW)], r_vm)
        pltpu.sync_copy(wflat_hbm.at[pl.ds(abase, APW)], w_vm)
        for i in range(APW // L):
            k = i // (TPW // L)
            l_v = (i % (TPW // L)) * L + lax.iota(jnp.int32, L)
            a_v = 2 * l_v + k
            e_v = plsc.load_gather(e_vm, [a_v])
            r_v = plsc.load_gather(r_vm, [a_v])
            pos_vm[pl.ds(i * L, L)] = plsc.load_gather(poff_vm, [e_v]) + r_v
        pltpu.async_copy(outp_hbm.at[pos_vm], rows_vm, sem1).wait()

        def body(l, carry):
            i0 = jnp.zeros((L,), jnp.int32) + 2 * l
            w0 = plsc.load_gather(w_vm, [i0])
            w1 = plsc.load_gather(w_vm, [i0 + 1])
            for v in range(D // L):
                sl = pl.ds(v * L, L)
                rows_vm[l, sl] = (rows_vm[l, sl] * w0
                                  + rows_vm[TPW + l, sl] * w1)
            return carry

        lax.fori_loop(0, TPW, body, 0)
        pltpu.sync_copy(rows_vm.at[pl.ds(0, TPW)], y_hbm.at[pl.ds(tbase, TPW)])

    return _sc_combine


def kernel(x, Wgate, Wg, Wu, Wd):
    xf = x.reshape(T, D)
    eidx, wtop, rank, poff16 = _router(xf, Wgate)

    eflat = eidx.reshape(A)
    rflat = rank.reshape(A)
    wflat = wtop.reshape(A)
    poff_flat = poff16.reshape(16)

    xs = _get_sc_dispatch()(xf, eflat, rflat, poff_flat)
    outp = _ffn_grouped(poff_flat, xs, Wg, Wu, Wd)
    y = _get_sc_combine()(outp, eflat, rflat, wflat, poff_flat)
    return y.reshape(B, S, D)
